# baseline (speedup 1.0000x reference)
import functools
import math

import jax
import jax.numpy as jnp
import numpy as np
from jax import lax
from jax.experimental import pallas as pl
from jax.experimental.pallas import tpu as pltpu
from jax.experimental.pallas import tpu_sc as plsc

N = 10000
E = 320000
D = 128
A = 16
S = 16
H = 8
AVG_NEI = 32.0

def _ssp_norm_const():
    xs = np.linspace(-12.0, 12.0, 200001)
    pdf = np.exp(-xs * xs / 2.0) / np.sqrt(2.0 * np.pi)
    vals = (np.logaddexp(0.0, xs) - np.log(2.0)) ** 2
    return 1.0 / np.sqrt(np.trapz(vals * pdf, xs))

SSP_C = float(_ssp_norm_const())

NC = 2
NS = 16
DH = D // NC
DP = DH // 2
EPS = E // NS
K = 100
KH = K // 2
NCHUNK = EPS // K
RPT = 624
REM = N - NS * RPT


BN = 1000
BE = 6400


def _lin1_body(ni_ref, w_ref, o_ref):
    y = jnp.dot(ni_ref[...], w_ref[0],
                preferred_element_type=jnp.float32) * (1.0 / math.sqrt(D))
    ye = y[:, :DP]
    yo = y[:, DP:]
    lo = lax.bitcast_convert_type(ye.astype(jnp.bfloat16),
                                  jnp.uint16).astype(jnp.uint32)
    hi = lax.bitcast_convert_type(yo.astype(jnp.bfloat16),
                                  jnp.uint16).astype(jnp.uint32)
    o_ref[0] = lax.bitcast_convert_type(lo | (hi << 16), jnp.int32)


def _lin1(node_input, W_lin1_h):
    return pl.pallas_call(
        _lin1_body,
        grid=(N // BN, NC),
        in_specs=[pl.BlockSpec((BN, D), lambda i, h: (i, 0)),
                  pl.BlockSpec((1, D, DH), lambda i, h: (h, 0, 0))],
        out_specs=pl.BlockSpec((1, BN, DP), lambda i, h: (h, i, 0)),
        out_shape=jax.ShapeDtypeStruct((NC, N, DP), jnp.int32),
    )(node_input, W_lin1_h)


def _wgt_body(est_ref, w1_ref, w2_ref, o_ref):
    x_t = lax.dot_general(w1_ref[...], est_ref[...],
                          (((0,), (0,)), ((), ())),
                          preferred_element_type=jnp.float32) * (
                              1.0 / math.sqrt(S))
    h_t = (jax.nn.softplus(x_t) - math.log(2.0)) * SSP_C
    w = lax.dot_general(h_t.astype(jnp.bfloat16),
                        w2_ref[...].astype(jnp.bfloat16),
                        (((0,), (0,)), ((), ())),
                        preferred_element_type=jnp.float32) * (
                            1.0 / math.sqrt(H))
    lo = lax.bitcast_convert_type(w[:BE // 2].astype(jnp.bfloat16),
                                  jnp.uint16).astype(jnp.uint32)
    hi = lax.bitcast_convert_type(w[BE // 2:].astype(jnp.bfloat16),
                                  jnp.uint16).astype(jnp.uint32)
    o_ref[...] = lax.bitcast_convert_type(lo | (hi << 16), jnp.int32)


def _edge_weights(est, W_fc1, W_fc2):
    return pl.pallas_call(
        _wgt_body,
        grid=(E // BE,),
        in_specs=[pl.BlockSpec((S, BE), lambda i: (0, i)),
                  pl.BlockSpec((S, H), lambda i: (0, 0)),
                  pl.BlockSpec((H, D), lambda i: (0, 0))],
        out_specs=pl.BlockSpec((BE // 2, D), lambda i: (i, 0)),
        out_shape=jax.ShapeDtypeStruct((E // 2, D), jnp.int32),
    )(est, W_fc1, W_fc2)


def _sc_term_body(ni_ref, na_ref, wsc_ref, o_ref):
    ni = ni_ref[...]
    na = na_ref[...]
    csc = 1.0 / math.sqrt(D * A)
    acc = na[:, 0:1] * (jnp.dot(ni, wsc_ref[0],
                                preferred_element_type=jnp.float32) * csc)
    for v in range(1, A):
        acc = acc + na[:, v:v + 1] * (
            jnp.dot(ni, wsc_ref[v], preferred_element_type=jnp.float32) * csc)
    o_ref[...] = acc


def _sc_term(node_input, node_attr, W_sc_t):
    return pl.pallas_call(
        _sc_term_body,
        grid=(N // BN,),
        in_specs=[pl.BlockSpec((BN, D), lambda i: (i, 0)),
                  pl.BlockSpec((BN, A), lambda i: (i, 0)),
                  pl.BlockSpec((A, D, D), lambda i: (0, 0, 0))],
        out_specs=pl.BlockSpec((BN, D), lambda i: (i, 0)),
        out_shape=jax.ShapeDtypeStruct((N, D), jnp.float32),
    )(node_input, node_attr, W_sc_t)


def _post_body(p_ref, sc_ref, wl2_ref, o_ref):
    cagg = 1.0 / (math.sqrt(D) * math.sqrt(AVG_NEI))
    o_ref[...] = (jnp.dot(p_ref[0], wl2_ref[0],
                          preferred_element_type=jnp.float32)
                  + jnp.dot(p_ref[1], wl2_ref[1],
                            preferred_element_type=jnp.float32)) * cagg + \
        sc_ref[...]


def _post(partials, sc_term, W_lin2_h):
    return pl.pallas_call(
        _post_body,
        grid=(N // BN,),
        in_specs=[pl.BlockSpec((NC, BN, DH), lambda i: (0, i, 0)),
                  pl.BlockSpec((BN, D), lambda i: (i, 0)),
                  pl.BlockSpec((NC, DH, D), lambda i: (0, 0, 0))],
        out_specs=pl.BlockSpec((BN, D), lambda i: (i, 0)),
        out_shape=jax.ShapeDtypeStruct((N, D), jnp.float32),
    )(partials, sc_term, W_lin2_h)


_sc_mesh = plsc.VectorSubcoreMesh(core_axis_name="c", subcore_axis_name="s")


@functools.partial(
    pl.kernel,
    out_type=jax.ShapeDtypeStruct((NC, N, DH), jnp.float32),
    mesh=_sc_mesh,
    compiler_params=pltpu.CompilerParams(use_tc_tiling_on_sc=False),
    scratch_types=[
        pltpu.VMEM((NCHUNK, K), jnp.int32),
        pltpu.VMEM((NCHUNK, K), jnp.int32),
        pltpu.VMEM((3, K, DP), jnp.int32),
        pltpu.VMEM((3, KH, D), jnp.int32),
        pltpu.VMEM((K, DH), jnp.float32),
        pltpu.VMEM_SHARED((N, DH), jnp.float32),
        pltpu.SemaphoreType.DMA,
        pltpu.SemaphoreType.DMA,
    ],
)
def _sc_scatter(nf_hbm, wgt_hbm, dst_hbm, src_hbm, zeros_hbm, zg_hbm,
                out_hbm, dst_v, src_v, gath_v, wv, ef_v, agg_sh, gsem, wsem):
    c = lax.axis_index("c")
    s = lax.axis_index("s")
    cbase = c * DH

    pltpu.sync_copy(zeros_hbm.at[pl.ds(s * RPT, RPT)],
                    agg_sh.at[pl.ds(s * RPT, RPT)])
    @pl.when(s == NS - 1)
    def _():
        pltpu.sync_copy(zeros_hbm.at[pl.ds(NS * RPT, REM)],
                        agg_sh.at[pl.ds(NS * RPT, REM)])

    pltpu.sync_copy(dst_hbm.at[s], dst_v)
    pltpu.sync_copy(src_hbm.at[s], src_v)
    plsc.subcore_barrier()

    ebase = s * EPS

    def loop_body(jj, carry):
        b = lax.rem(jj, 3)
        jc = jnp.minimum(jj, NCHUNK - 1)
        pltpu.async_copy(nf_hbm.at[c].at[dst_v.at[jc]], gath_v.at[b], gsem)

        @pl.when(jj < NCHUNK)
        def _():
            pltpu.async_copy(
                wgt_hbm.at[pl.ds((ebase + jj * K) // 2, KH)], wv.at[b],
                wsem)

        @pl.when(jj > 1)
        def _():
            j = jj - 2
            bb = lax.rem(j, 3)
            pltpu.make_async_copy(zg_hbm, gath_v.at[bb], gsem).wait()
            pltpu.make_async_copy(wgt_hbm.at[pl.ds(0, KH)], wv.at[bb],
                                  wsem).wait()

            def mul_body(r, carry2):
                e0 = r
                e1 = KH + r
                for g in range(DP // 16):
                    w0 = gath_v[bb, e0, pl.ds(g * 16, 16)]
                    w1 = gath_v[bb, e1, pl.ds(g * 16, 16)]
                    f0e = lax.bitcast_convert_type(
                        lax.shift_left(w0, 16), jnp.float32)
                    f0o = lax.bitcast_convert_type(
                        w0 & jnp.int32(-65536), jnp.float32)
                    f1e = lax.bitcast_convert_type(
                        lax.shift_left(w1, 16), jnp.float32)
                    f1o = lax.bitcast_convert_type(
                        w1 & jnp.int32(-65536), jnp.float32)
                    wwa = wv[bb, r, pl.ds(cbase + g * 32, 16)]
                    wwb = wv[bb, r, pl.ds(cbase + g * 32 + 16, 16)]
                    w0e = lax.bitcast_convert_type(
                        lax.shift_left(wwa, 16), jnp.float32)
                    w1e = lax.bitcast_convert_type(
                        wwa & jnp.int32(-65536), jnp.float32)
                    w0o = lax.bitcast_convert_type(
                        lax.shift_left(wwb, 16), jnp.float32)
                    w1o = lax.bitcast_convert_type(
                        wwb & jnp.int32(-65536), jnp.float32)
                    ef_v[e0, pl.ds(g * 32, 16)] = f0e * w0e
                    ef_v[e0, pl.ds(g * 32 + 16, 16)] = f0o * w0o
                    ef_v[e1, pl.ds(g * 32, 16)] = f1e * w1e
                    ef_v[e1, pl.ds(g * 32 + 16, 16)] = f1o * w1o
                return carry2

            lax.fori_loop(0, KH, mul_body, 0, unroll=2)
            pltpu.sync_copy(ef_v, agg_sh.at[src_v.at[j]], add=True)

        return carry

    lax.fori_loop(0, NCHUNK + 2, loop_body, 0)
    pltpu.make_async_copy(zg_hbm, gath_v.at[0], gsem).wait()
    pltpu.make_async_copy(zg_hbm, gath_v.at[1], gsem).wait()

    plsc.subcore_barrier()
    pltpu.sync_copy(agg_sh.at[pl.ds(s * RPT, RPT)],
                    out_hbm.at[c, pl.ds(s * RPT, RPT)])
    @pl.when(s == NS - 1)
    def _():
        pltpu.sync_copy(agg_sh.at[pl.ds(NS * RPT, REM)],
                        out_hbm.at[c, pl.ds(NS * RPT, REM)])


_PG = np.arange(32).reshape(16, 2).T.reshape(32)
_PERM = np.concatenate([g * 32 + _PG for g in range(D // 32)])
_PACKSRC = np.concatenate(
    [h * DH + np.concatenate([np.arange(0, DH, 2), np.arange(1, DH, 2)])
     for h in range(NC)])


def kernel(node_input, node_attr, edge_attr, edge_scalars, W_lin1, W_fc1,
           W_fc2, W_lin2, W_sc, edge_src, edge_dst):
    W_lin1_h = W_lin1[:, _PACKSRC].reshape(D, NC, DH).transpose(1, 0, 2)
    nf = _lin1(node_input, W_lin1_h)
    wgt = _edge_weights(edge_scalars.T, W_fc1, W_fc2[:, _PERM])
    def _slots(x):
        return (x.reshape(E // BE, 2, BE // (2 * KH), KH)
                .transpose(0, 2, 1, 3).reshape(NS, NCHUNK, K))
    dst3 = _slots(edge_dst)
    src3 = _slots(edge_src)
    zeros = jnp.zeros((N, DH), jnp.float32)
    zgath = jnp.zeros((K, DP), jnp.int32)
    partials = _sc_scatter(nf, wgt, dst3, src3, zeros, zgath)
    W_lin2_h = W_lin2[_PERM, :].reshape(NC, DH, D)
    W_sc_t = jnp.transpose(W_sc, (1, 0, 2))
    sct = _sc_term(node_input, node_attr, W_sc_t)
    return _post(partials, sct, W_lin2_h)


# speedup vs baseline: 1.0032x; 1.0032x over previous
import functools
import math

import jax
import jax.numpy as jnp
import numpy as np
from jax import lax
from jax.experimental import pallas as pl
from jax.experimental.pallas import tpu as pltpu
from jax.experimental.pallas import tpu_sc as plsc

N = 10000
E = 320000
D = 128
A = 16
S = 16
H = 8
AVG_NEI = 32.0

def _ssp_norm_const():
    xs = np.linspace(-12.0, 12.0, 200001)
    pdf = np.exp(-xs * xs / 2.0) / np.sqrt(2.0 * np.pi)
    vals = (np.logaddexp(0.0, xs) - np.log(2.0)) ** 2
    return 1.0 / np.sqrt(np.trapz(vals * pdf, xs))

SSP_C = float(_ssp_norm_const())

NC = 2
NS = 16
DH = D // NC
DP = DH // 2
EPS = E // NS
K = 100
KH = K // 2
NCHUNK = EPS // K
RPT = 624
REM = N - NS * RPT


BN = 1000
BE = 6400


def _lin1_body(ni_ref, w_ref, o_ref):
    y = jnp.dot(ni_ref[...], w_ref[0],
                preferred_element_type=jnp.float32) * (1.0 / math.sqrt(D))
    ye = y[:, :DP]
    yo = y[:, DP:]
    lo = lax.bitcast_convert_type(ye.astype(jnp.bfloat16),
                                  jnp.uint16).astype(jnp.uint32)
    hi = lax.bitcast_convert_type(yo.astype(jnp.bfloat16),
                                  jnp.uint16).astype(jnp.uint32)
    o_ref[0] = lax.bitcast_convert_type(lo | (hi << 16), jnp.int32)


def _lin1(node_input, W_lin1_h):
    return pl.pallas_call(
        _lin1_body,
        grid=(N // BN, NC),
        in_specs=[pl.BlockSpec((BN, D), lambda i, h: (i, 0)),
                  pl.BlockSpec((1, D, DH), lambda i, h: (h, 0, 0))],
        out_specs=pl.BlockSpec((1, BN, DP), lambda i, h: (h, i, 0)),
        out_shape=jax.ShapeDtypeStruct((NC, N, DP), jnp.int32),
    )(node_input, W_lin1_h)


def _wgt_body(est_ref, w1_ref, w2_ref, o_ref):
    x_t = lax.dot_general(w1_ref[...], est_ref[...],
                          (((0,), (0,)), ((), ())),
                          preferred_element_type=jnp.float32) * (
                              1.0 / math.sqrt(S))
    h_t = (jax.nn.softplus(x_t) - math.log(2.0)) * SSP_C
    w = lax.dot_general(h_t.astype(jnp.bfloat16),
                        w2_ref[...].astype(jnp.bfloat16),
                        (((0,), (0,)), ((), ())),
                        preferred_element_type=jnp.float32) * (
                            1.0 / math.sqrt(H))
    lo = lax.bitcast_convert_type(w[:BE // 2].astype(jnp.bfloat16),
                                  jnp.uint16).astype(jnp.uint32)
    hi = lax.bitcast_convert_type(w[BE // 2:].astype(jnp.bfloat16),
                                  jnp.uint16).astype(jnp.uint32)
    o_ref[...] = lax.bitcast_convert_type(lo | (hi << 16), jnp.int32)


def _edge_weights(est, W_fc1, W_fc2):
    return pl.pallas_call(
        _wgt_body,
        grid=(E // BE,),
        in_specs=[pl.BlockSpec((S, BE), lambda i: (0, i)),
                  pl.BlockSpec((S, H), lambda i: (0, 0)),
                  pl.BlockSpec((H, D), lambda i: (0, 0))],
        out_specs=pl.BlockSpec((BE // 2, D), lambda i: (i, 0)),
        out_shape=jax.ShapeDtypeStruct((E // 2, D), jnp.int32),
    )(est, W_fc1, W_fc2)


def _sc_term_body(ni_ref, na_ref, wsc_ref, o_ref):
    ni = ni_ref[...]
    na = na_ref[...]
    csc = 1.0 / math.sqrt(D * A)
    acc = na[:, 0:1] * (jnp.dot(ni, wsc_ref[0],
                                preferred_element_type=jnp.float32) * csc)
    for v in range(1, A):
        acc = acc + na[:, v:v + 1] * (
            jnp.dot(ni, wsc_ref[v], preferred_element_type=jnp.float32) * csc)
    o_ref[...] = acc


def _sc_term(node_input, node_attr, W_sc_t):
    return pl.pallas_call(
        _sc_term_body,
        grid=(N // BN,),
        in_specs=[pl.BlockSpec((BN, D), lambda i: (i, 0)),
                  pl.BlockSpec((BN, A), lambda i: (i, 0)),
                  pl.BlockSpec((A, D, D), lambda i: (0, 0, 0))],
        out_specs=pl.BlockSpec((BN, D), lambda i: (i, 0)),
        out_shape=jax.ShapeDtypeStruct((N, D), jnp.float32),
    )(node_input, node_attr, W_sc_t)


def _post_body(p_ref, sc_ref, wl2_ref, o_ref):
    cagg = 1.0 / (math.sqrt(D) * math.sqrt(AVG_NEI))
    o_ref[...] = (jnp.dot(p_ref[0], wl2_ref[0],
                          preferred_element_type=jnp.float32)
                  + jnp.dot(p_ref[1], wl2_ref[1],
                            preferred_element_type=jnp.float32)) * cagg + \
        sc_ref[...]


def _post(partials, sc_term, W_lin2_h):
    return pl.pallas_call(
        _post_body,
        grid=(N // BN,),
        in_specs=[pl.BlockSpec((NC, BN, DH), lambda i: (0, i, 0)),
                  pl.BlockSpec((BN, D), lambda i: (i, 0)),
                  pl.BlockSpec((NC, DH, D), lambda i: (0, 0, 0))],
        out_specs=pl.BlockSpec((BN, D), lambda i: (i, 0)),
        out_shape=jax.ShapeDtypeStruct((N, D), jnp.float32),
    )(partials, sc_term, W_lin2_h)


_sc_mesh = plsc.VectorSubcoreMesh(core_axis_name="c", subcore_axis_name="s")


@functools.partial(
    pl.kernel,
    out_type=jax.ShapeDtypeStruct((NC, N, DH), jnp.float32),
    mesh=_sc_mesh,
    compiler_params=pltpu.CompilerParams(use_tc_tiling_on_sc=False),
    scratch_types=[
        pltpu.VMEM((NCHUNK, K), jnp.int32),
        pltpu.VMEM((NCHUNK, K), jnp.int32),
        pltpu.VMEM((2, K, DP), jnp.int32),
        pltpu.VMEM((2, KH, D), jnp.int32),
        pltpu.VMEM((K, DH), jnp.float32),
        pltpu.VMEM_SHARED((N, DH), jnp.float32),
        pltpu.SemaphoreType.DMA,
        pltpu.SemaphoreType.DMA,
    ],
)
def _sc_scatter(nf_hbm, wgt_hbm, dst_hbm, src_hbm, zeros_hbm, zg_hbm,
                out_hbm, dst_v, src_v, gath_v, wv, ef_v, agg_sh, gsem, wsem):
    c = lax.axis_index("c")
    s = lax.axis_index("s")
    cbase = c * DH

    pltpu.sync_copy(zeros_hbm.at[pl.ds(s * RPT, RPT)],
                    agg_sh.at[pl.ds(s * RPT, RPT)])
    @pl.when(s == NS - 1)
    def _():
        pltpu.sync_copy(zeros_hbm.at[pl.ds(NS * RPT, REM)],
                        agg_sh.at[pl.ds(NS * RPT, REM)])

    pltpu.sync_copy(dst_hbm.at[s], dst_v)
    pltpu.sync_copy(src_hbm.at[s], src_v)
    plsc.subcore_barrier()

    ebase = s * EPS

    def loop_body(jj, carry):
        b = jj & 1
        jc = jnp.minimum(jj, NCHUNK - 1)
        pltpu.async_copy(nf_hbm.at[c].at[dst_v.at[jc]], gath_v.at[b], gsem)

        @pl.when(jj < NCHUNK)
        def _():
            pltpu.async_copy(
                wgt_hbm.at[pl.ds((ebase + jj * K) // 2, KH)], wv.at[b],
                wsem)

        @pl.when(jj > 0)
        def _():
            j = jj - 1
            bb = j & 1
            pltpu.make_async_copy(zg_hbm, gath_v.at[bb], gsem).wait()
            pltpu.make_async_copy(wgt_hbm.at[pl.ds(0, KH)], wv.at[bb],
                                  wsem).wait()

            def mul_body(r, carry2):
                e0 = r
                e1 = KH + r
                for g in range(DP // 16):
                    w0 = gath_v[bb, e0, pl.ds(g * 16, 16)]
                    w1 = gath_v[bb, e1, pl.ds(g * 16, 16)]
                    f0e = lax.bitcast_convert_type(
                        lax.shift_left(w0, 16), jnp.float32)
                    f0o = lax.bitcast_convert_type(
                        w0 & jnp.int32(-65536), jnp.float32)
                    f1e = lax.bitcast_convert_type(
                        lax.shift_left(w1, 16), jnp.float32)
                    f1o = lax.bitcast_convert_type(
                        w1 & jnp.int32(-65536), jnp.float32)
                    wwa = wv[bb, r, pl.ds(cbase + g * 32, 16)]
                    wwb = wv[bb, r, pl.ds(cbase + g * 32 + 16, 16)]
                    w0e = lax.bitcast_convert_type(
                        lax.shift_left(wwa, 16), jnp.float32)
                    w1e = lax.bitcast_convert_type(
                        wwa & jnp.int32(-65536), jnp.float32)
                    w0o = lax.bitcast_convert_type(
                        lax.shift_left(wwb, 16), jnp.float32)
                    w1o = lax.bitcast_convert_type(
                        wwb & jnp.int32(-65536), jnp.float32)
                    ef_v[e0, pl.ds(g * 32, 16)] = f0e * w0e
                    ef_v[e0, pl.ds(g * 32 + 16, 16)] = f0o * w0o
                    ef_v[e1, pl.ds(g * 32, 16)] = f1e * w1e
                    ef_v[e1, pl.ds(g * 32 + 16, 16)] = f1o * w1o
                return carry2

            lax.fori_loop(0, KH, mul_body, 0, unroll=2)
            pltpu.sync_copy(ef_v, agg_sh.at[src_v.at[j]], add=True)

        return carry

    lax.fori_loop(0, NCHUNK + 1, loop_body, 0)
    pltpu.make_async_copy(zg_hbm, gath_v.at[0], gsem).wait()

    plsc.subcore_barrier()
    pltpu.sync_copy(agg_sh.at[pl.ds(s * RPT, RPT)],
                    out_hbm.at[c, pl.ds(s * RPT, RPT)])
    @pl.when(s == NS - 1)
    def _():
        pltpu.sync_copy(agg_sh.at[pl.ds(NS * RPT, REM)],
                        out_hbm.at[c, pl.ds(NS * RPT, REM)])


_PG = np.arange(32).reshape(16, 2).T.reshape(32)
_PERM = np.concatenate([g * 32 + _PG for g in range(D // 32)])
_PACKSRC = np.concatenate(
    [h * DH + np.concatenate([np.arange(0, DH, 2), np.arange(1, DH, 2)])
     for h in range(NC)])


def kernel(node_input, node_attr, edge_attr, edge_scalars, W_lin1, W_fc1,
           W_fc2, W_lin2, W_sc, edge_src, edge_dst):
    W_lin1_h = W_lin1[:, _PACKSRC].reshape(D, NC, DH).transpose(1, 0, 2)
    nf = _lin1(node_input, W_lin1_h)
    wgt = _edge_weights(edge_scalars.T, W_fc1, W_fc2[:, _PERM])
    def _slots(x):
        return (x.reshape(E // BE, 2, BE // (2 * KH), KH)
                .transpose(0, 2, 1, 3).reshape(NS, NCHUNK, K))
    dst3 = _slots(edge_dst)
    src3 = _slots(edge_src)
    zeros = jnp.zeros((N, DH), jnp.float32)
    zgath = jnp.zeros((K, DP), jnp.int32)
    partials = _sc_scatter(nf, wgt, dst3, src3, zeros, zgath)
    W_lin2_h = W_lin2[_PERM, :].reshape(NC, DH, D)
    W_sc_t = jnp.transpose(W_sc, (1, 0, 2))
    sct = _sc_term(node_input, node_attr, W_sc_t)
    return _post(partials, sct, W_lin2_h)
